# Initial kernel scaffold; baseline (speedup 1.0000x reference)
import functools

import jax
import jax.numpy as jnp
from jax import lax
from jax.experimental import pallas as pl
from jax.experimental.pallas import tpu as pltpu
from jax.experimental.pallas import tpu_sc as plsc

N = 10000
E = 320000
D = 128
NC = 2
NS = 16
NW = NC * NS
EPT = E // NW
CHUNK = 128
NFULL = EPT // CHUNK
REM = EPT - NFULL * CHUNK
RPT = N // NS
ZR = 125


def _sc_hop(h, src, dst):
    mesh = plsc.VectorSubcoreMesh(core_axis_name="c", subcore_axis_name="s")

    @functools.partial(
        pl.kernel,
        mesh=mesh,
        out_type=jax.ShapeDtypeStruct((NC, N, D), jnp.float32),
        scratch_types=[
            pltpu.VMEM((CHUNK,), jnp.int32),
            pltpu.VMEM((CHUNK,), jnp.int32),
            pltpu.VMEM((CHUNK, D), jnp.float32),
            pltpu.VMEM((REM,), jnp.int32),
            pltpu.VMEM((REM,), jnp.int32),
            pltpu.VMEM((REM, D), jnp.float32),
            pltpu.VMEM((ZR, D), jnp.float32),
            pltpu.VMEM_SHARED((N, D), jnp.float32),
            pltpu.SemaphoreType.DMA,
        ],
    )
    def hop(h_hbm, src_hbm, dst_hbm, out_hbm,
            src_v, dst_v, rows_v, src_r, dst_r, rows_r, zbuf, agg, sem):
        c = lax.axis_index("c")
        s = lax.axis_index("s")
        wid = s * NC + c

        def zbody(i, carry):
            for k in range(D // 16):
                zbuf[i, pl.ds(k * 16, 16)] = jnp.zeros((16,), jnp.float32)
            return carry
        lax.fori_loop(0, ZR, zbody, 0)
        row0 = s * RPT
        for k in range(RPT // ZR):
            pltpu.sync_copy(zbuf, agg.at[pl.ds(row0 + k * ZR, ZR)])
        plsc.subcore_barrier()

        def body(j, carry):
            base = wid * EPT + j * CHUNK
            pltpu.sync_copy(src_hbm.at[pl.ds(base, CHUNK)], src_v)
            pltpu.sync_copy(dst_hbm.at[pl.ds(base, CHUNK)], dst_v)
            pltpu.async_copy(h_hbm.at[src_v], rows_v, sem).wait()
            pltpu.sync_copy(rows_v, agg.at[dst_v], add=True)
            return carry
        lax.fori_loop(0, NFULL, body, 0)

        rbase = wid * EPT + NFULL * CHUNK
        pltpu.sync_copy(src_hbm.at[pl.ds(rbase, REM)], src_r)
        pltpu.sync_copy(dst_hbm.at[pl.ds(rbase, REM)], dst_r)
        pltpu.async_copy(h_hbm.at[src_r], rows_r, sem).wait()
        pltpu.sync_copy(rows_r, agg.at[dst_r], add=True)

        plsc.subcore_barrier()
        pltpu.sync_copy(agg.at[pl.ds(row0, RPT)],
                        out_hbm.at[c, pl.ds(row0, RPT)])

    return hop(h, src, dst)


_DOT = dict(preferred_element_type=jnp.float32,
            precision=lax.Precision.HIGHEST)
_BR = 1000


def _tc_mid(p, h, wr, wt):
    def body(p_ref, h_ref, wr_ref, wt_ref, o_ref):
        agg = p_ref[0] + p_ref[1]
        y = lax.dot_general(agg, wr_ref[...], (((1,), (1,)), ((), ())), **_DOT)
        y = y + lax.dot_general(h_ref[...], wt_ref[...],
                                (((1,), (1,)), ((), ())), **_DOT)
        o_ref[...] = jnp.maximum(y, 0.0)

    return pl.pallas_call(
        body,
        grid=(N // _BR,),
        in_specs=[
            pl.BlockSpec((NC, _BR, D), lambda i: (0, i, 0)),
            pl.BlockSpec((_BR, D), lambda i: (i, 0)),
            pl.BlockSpec((D, D), lambda i: (0, 0)),
            pl.BlockSpec((D, D), lambda i: (0, 0)),
        ],
        out_specs=pl.BlockSpec((_BR, D), lambda i: (i, 0)),
        out_shape=jax.ShapeDtypeStruct((N, D), jnp.float32),
    )(p, h, wr, wt)


def _tc_final(p, h, wr_s, wt_s):
    ksel = wr_s.shape[0]

    def body(p_ref, h_ref, wr_ref, wt_ref, o_ref):
        agg = p_ref[0] + p_ref[1]
        y = lax.dot_general(agg, wr_ref[...], (((1,), (1,)), ((), ())), **_DOT)
        y = y + lax.dot_general(h_ref[...], wt_ref[...],
                                (((1,), (1,)), ((), ())), **_DOT)
        o_ref[...] = jnp.sum(jnp.maximum(y, 0.0), axis=1, keepdims=True)

    return pl.pallas_call(
        body,
        grid=(N // _BR,),
        in_specs=[
            pl.BlockSpec((NC, _BR, D), lambda i: (0, i, 0)),
            pl.BlockSpec((_BR, D), lambda i: (i, 0)),
            pl.BlockSpec((ksel, D), lambda i: (0, 0)),
            pl.BlockSpec((ksel, D), lambda i: (0, 0)),
        ],
        out_specs=pl.BlockSpec((_BR, 1), lambda i: (i, 0)),
        out_shape=jax.ShapeDtypeStruct((N, 1), jnp.float32),
    )(p, h, wr_s, wt_s)


def kernel(x, edge_index, batch, W_rel, W_root):
    del batch
    src = edge_index[0]
    dst = edge_index[1]
    step = 4
    wr_s = W_rel[step - 1::step]
    wt_s = W_root[step - 1::step]

    p1 = _sc_hop(x, src, dst)
    h1 = _tc_mid(p1, x, W_rel, W_root)
    p2 = _sc_hop(h1, src, dst)
    out = _tc_final(p2, h1, wr_s, wt_s)
    return out[:, 0]


# speedup vs baseline: 6.3149x; 6.3149x over previous
import functools

import jax
import jax.numpy as jnp
from jax import lax
from jax.experimental import pallas as pl
from jax.experimental.pallas import tpu as pltpu
from jax.experimental.pallas import tpu_sc as plsc

N = 10000
E = 320000
D = 128
NC = 2
NS = 16
NW = NC * NS
EPT = E // NW
CHUNK = 128
NFULL = EPT // CHUNK
REM = EPT - NFULL * CHUNK
RPT = 624
TAIL = N - NS * RPT
ZR = 104


def _sc_hop(h, src, dst):
    mesh = plsc.VectorSubcoreMesh(core_axis_name="c", subcore_axis_name="s",
                                  num_cores=NC, num_subcores=NS)

    @functools.partial(
        pl.kernel,
        mesh=mesh,
        out_type=jax.ShapeDtypeStruct((NC, N, D), jnp.float32),
        scratch_types=[
            pltpu.VMEM((CHUNK,), jnp.int32),
            pltpu.VMEM((CHUNK,), jnp.int32),
            pltpu.VMEM((CHUNK, D), jnp.float32),
            pltpu.VMEM((REM,), jnp.int32),
            pltpu.VMEM((REM,), jnp.int32),
            pltpu.VMEM((REM, D), jnp.float32),
            pltpu.VMEM((ZR, D), jnp.float32),
            pltpu.VMEM_SHARED((N, D), jnp.float32),
            pltpu.SemaphoreType.DMA,
        ],
    )
    def hop(h_hbm, src_hbm, dst_hbm, out_hbm,
            src_v, dst_v, rows_v, src_r, dst_r, rows_r, zbuf, agg, sem):
        c = lax.axis_index("c")
        s = lax.axis_index("s")
        wid = s * NC + c

        def zbody(i, carry):
            for k in range(D // 16):
                zbuf[i, pl.ds(k * 16, 16)] = jnp.zeros((16,), jnp.float32)
            return carry
        lax.fori_loop(0, ZR, zbody, 0)
        row0 = s * RPT
        for k in range(RPT // ZR):
            pltpu.sync_copy(zbuf, agg.at[pl.ds(row0 + k * ZR, ZR)])

        @pl.when(s == NS - 1)
        def _zero_tail():
            pltpu.sync_copy(zbuf.at[pl.ds(0, TAIL)],
                            agg.at[pl.ds(NS * RPT, TAIL)])
        plsc.subcore_barrier()

        def body(j, carry):
            base = wid * EPT + j * CHUNK
            pltpu.sync_copy(src_hbm.at[pl.ds(base, CHUNK)], src_v)
            pltpu.sync_copy(dst_hbm.at[pl.ds(base, CHUNK)], dst_v)
            pltpu.async_copy(h_hbm.at[src_v], rows_v, sem).wait()
            pltpu.sync_copy(rows_v, agg.at[dst_v], add=True)
            return carry
        lax.fori_loop(0, NFULL, body, 0)

        rbase = wid * EPT + NFULL * CHUNK
        pltpu.sync_copy(src_hbm.at[pl.ds(rbase, REM)], src_r)
        pltpu.sync_copy(dst_hbm.at[pl.ds(rbase, REM)], dst_r)
        pltpu.async_copy(h_hbm.at[src_r], rows_r, sem).wait()
        pltpu.sync_copy(rows_r, agg.at[dst_r], add=True)

        plsc.subcore_barrier()
        pltpu.sync_copy(agg.at[pl.ds(row0, RPT)],
                        out_hbm.at[c, pl.ds(row0, RPT)])

        @pl.when(s == NS - 1)
        def _copy_tail():
            pltpu.sync_copy(agg.at[pl.ds(NS * RPT, TAIL)],
                            out_hbm.at[c, pl.ds(NS * RPT, TAIL)])

    return hop(h, src, dst)


_DOT = dict(preferred_element_type=jnp.float32,
            precision=lax.Precision.HIGHEST)
_BR = 1000


def _tc_mid(p, h, wr, wt):
    def body(p_ref, h_ref, wr_ref, wt_ref, o_ref):
        agg = p_ref[0] + p_ref[1]
        y = lax.dot_general(agg, wr_ref[...], (((1,), (1,)), ((), ())), **_DOT)
        y = y + lax.dot_general(h_ref[...], wt_ref[...],
                                (((1,), (1,)), ((), ())), **_DOT)
        o_ref[...] = jnp.maximum(y, 0.0)

    return pl.pallas_call(
        body,
        grid=(N // _BR,),
        in_specs=[
            pl.BlockSpec((NC, _BR, D), lambda i: (0, i, 0)),
            pl.BlockSpec((_BR, D), lambda i: (i, 0)),
            pl.BlockSpec((D, D), lambda i: (0, 0)),
            pl.BlockSpec((D, D), lambda i: (0, 0)),
        ],
        out_specs=pl.BlockSpec((_BR, D), lambda i: (i, 0)),
        out_shape=jax.ShapeDtypeStruct((N, D), jnp.float32),
    )(p, h, wr, wt)


def _tc_final(p, h, wr_s, wt_s):
    ksel = wr_s.shape[0]

    def body(p_ref, h_ref, wr_ref, wt_ref, o_ref):
        agg = p_ref[0] + p_ref[1]
        y = lax.dot_general(agg, wr_ref[...], (((1,), (1,)), ((), ())), **_DOT)
        y = y + lax.dot_general(h_ref[...], wt_ref[...],
                                (((1,), (1,)), ((), ())), **_DOT)
        o_ref[...] = jnp.sum(jnp.maximum(y, 0.0), axis=1, keepdims=True)

    return pl.pallas_call(
        body,
        grid=(N // _BR,),
        in_specs=[
            pl.BlockSpec((NC, _BR, D), lambda i: (0, i, 0)),
            pl.BlockSpec((_BR, D), lambda i: (i, 0)),
            pl.BlockSpec((ksel, D), lambda i: (0, 0)),
            pl.BlockSpec((ksel, D), lambda i: (0, 0)),
        ],
        out_specs=pl.BlockSpec((_BR, 1), lambda i: (i, 0)),
        out_shape=jax.ShapeDtypeStruct((N, 1), jnp.float32),
    )(p, h, wr_s, wt_s)


def kernel(x, edge_index, batch, W_rel, W_root):
    del batch
    src = edge_index[0]
    dst = edge_index[1]
    step = 4
    wr_s = W_rel[step - 1::step]
    wt_s = W_root[step - 1::step]

    p1 = _sc_hop(x, src, dst)
    h1 = _tc_mid(p1, x, W_rel, W_root)
    p2 = _sc_hop(h1, src, dst)
    out = _tc_final(p2, h1, wr_s, wt_s)
    return out[:, 0]


# speedup vs baseline: 8.7295x; 1.3824x over previous
import functools

import jax
import jax.numpy as jnp
from jax import lax
from jax.experimental import pallas as pl
from jax.experimental.pallas import tpu as pltpu
from jax.experimental.pallas import tpu_sc as plsc

N = 10000
E = 320000
D = 128
NC = 2
NS = 16
NW = NC * NS
EPT = E // NW
CHUNK = 128
NFULL = EPT // CHUNK
REM = EPT - NFULL * CHUNK
RPT = 624
TAIL = N - NS * RPT
ZR = 104


def _sc_hop(h, src, dst):
    mesh = plsc.VectorSubcoreMesh(core_axis_name="c", subcore_axis_name="s",
                                  num_cores=NC, num_subcores=NS)

    @functools.partial(
        pl.kernel,
        mesh=mesh,
        out_type=jax.ShapeDtypeStruct((NC, N, D), jnp.float32),
        scratch_types=[
            pltpu.VMEM((CHUNK,), jnp.int32),
            pltpu.VMEM((CHUNK,), jnp.int32),
            pltpu.VMEM((CHUNK, D), jnp.float32),
            pltpu.VMEM((CHUNK,), jnp.int32),
            pltpu.VMEM((CHUNK,), jnp.int32),
            pltpu.VMEM((CHUNK, D), jnp.float32),
            pltpu.VMEM((REM,), jnp.int32),
            pltpu.VMEM((REM,), jnp.int32),
            pltpu.VMEM((REM, D), jnp.float32),
            pltpu.VMEM((ZR, D), jnp.float32),
            pltpu.VMEM_SHARED((N, D), jnp.float32),
            pltpu.SemaphoreType.DMA,
            pltpu.SemaphoreType.DMA,
            pltpu.SemaphoreType.DMA,
            pltpu.SemaphoreType.DMA,
            pltpu.SemaphoreType.DMA,
            pltpu.SemaphoreType.DMA,
            pltpu.SemaphoreType.DMA,
        ],
    )
    def hop(h_hbm, src_hbm, dst_hbm, out_hbm,
            src0, dst0, rows0, src1, dst1, rows1, src_r, dst_r, rows_r,
            zbuf, agg, isem0, gsem0, ssem0, isem1, gsem1, ssem1, sem):
        c = lax.axis_index("c")
        s = lax.axis_index("s")
        wid = s * NC + c
        bufs = ((src0, dst0, rows0, isem0, gsem0, ssem0),
                (src1, dst1, rows1, isem1, gsem1, ssem1))

        def zbody(i, carry):
            for k in range(D // 16):
                zbuf[i, pl.ds(k * 16, 16)] = jnp.zeros((16,), jnp.float32)
            return carry
        lax.fori_loop(0, ZR, zbody, 0)
        row0 = s * RPT
        for k in range(RPT // ZR):
            pltpu.sync_copy(zbuf, agg.at[pl.ds(row0 + k * ZR, ZR)])

        @pl.when(s == NS - 1)
        def _zero_tail():
            pltpu.sync_copy(zbuf.at[pl.ds(0, TAIL)],
                            agg.at[pl.ds(NS * RPT, TAIL)])
        plsc.subcore_barrier()

        def body(i, carry):
            base0 = wid * EPT + (2 * i) * CHUNK
            base1 = base0 + CHUNK
            i0s = pltpu.async_copy(src_hbm.at[pl.ds(base0, CHUNK)], src0, isem0)
            i0d = pltpu.async_copy(dst_hbm.at[pl.ds(base0, CHUNK)], dst0, isem0)
            i1s = pltpu.async_copy(src_hbm.at[pl.ds(base1, CHUNK)], src1, isem1)
            i1d = pltpu.async_copy(dst_hbm.at[pl.ds(base1, CHUNK)], dst1, isem1)
            i0s.wait()
            i0d.wait()
            g0 = pltpu.async_copy(h_hbm.at[src0], rows0, gsem0)
            i1s.wait()
            i1d.wait()
            g1 = pltpu.async_copy(h_hbm.at[src1], rows1, gsem1)
            g0.wait()
            s0 = pltpu.async_copy(rows0, agg.at[dst0], ssem0, add=True)
            g1.wait()
            s1 = pltpu.async_copy(rows1, agg.at[dst1], ssem1, add=True)
            s0.wait()
            s1.wait()
            return carry
        lax.fori_loop(0, NFULL // 2, body, 0)

        rbase = wid * EPT + NFULL * CHUNK
        pltpu.sync_copy(src_hbm.at[pl.ds(rbase, REM)], src_r)
        pltpu.sync_copy(dst_hbm.at[pl.ds(rbase, REM)], dst_r)
        pltpu.async_copy(h_hbm.at[src_r], rows_r, sem).wait()
        pltpu.sync_copy(rows_r, agg.at[dst_r], add=True)

        plsc.subcore_barrier()
        pltpu.sync_copy(agg.at[pl.ds(row0, RPT)],
                        out_hbm.at[c, pl.ds(row0, RPT)])

        @pl.when(s == NS - 1)
        def _copy_tail():
            pltpu.sync_copy(agg.at[pl.ds(NS * RPT, TAIL)],
                            out_hbm.at[c, pl.ds(NS * RPT, TAIL)])

    return hop(h, src, dst)


_DOT = dict(preferred_element_type=jnp.float32,
            precision=lax.Precision.HIGHEST)
_BR = 1000


def _tc_mid(p, h, wr, wt):
    def body(p_ref, h_ref, wr_ref, wt_ref, o_ref):
        agg = p_ref[0] + p_ref[1]
        y = lax.dot_general(agg, wr_ref[...], (((1,), (1,)), ((), ())), **_DOT)
        y = y + lax.dot_general(h_ref[...], wt_ref[...],
                                (((1,), (1,)), ((), ())), **_DOT)
        o_ref[...] = jnp.maximum(y, 0.0)

    return pl.pallas_call(
        body,
        grid=(N // _BR,),
        in_specs=[
            pl.BlockSpec((NC, _BR, D), lambda i: (0, i, 0)),
            pl.BlockSpec((_BR, D), lambda i: (i, 0)),
            pl.BlockSpec((D, D), lambda i: (0, 0)),
            pl.BlockSpec((D, D), lambda i: (0, 0)),
        ],
        out_specs=pl.BlockSpec((_BR, D), lambda i: (i, 0)),
        out_shape=jax.ShapeDtypeStruct((N, D), jnp.float32),
    )(p, h, wr, wt)


def _tc_final(p, h, wr_s, wt_s):
    ksel = wr_s.shape[0]

    def body(p_ref, h_ref, wr_ref, wt_ref, o_ref):
        agg = p_ref[0] + p_ref[1]
        y = lax.dot_general(agg, wr_ref[...], (((1,), (1,)), ((), ())), **_DOT)
        y = y + lax.dot_general(h_ref[...], wt_ref[...],
                                (((1,), (1,)), ((), ())), **_DOT)
        o_ref[...] = jnp.sum(jnp.maximum(y, 0.0), axis=1, keepdims=True)

    return pl.pallas_call(
        body,
        grid=(N // _BR,),
        in_specs=[
            pl.BlockSpec((NC, _BR, D), lambda i: (0, i, 0)),
            pl.BlockSpec((_BR, D), lambda i: (i, 0)),
            pl.BlockSpec((ksel, D), lambda i: (0, 0)),
            pl.BlockSpec((ksel, D), lambda i: (0, 0)),
        ],
        out_specs=pl.BlockSpec((_BR, 1), lambda i: (i, 0)),
        out_shape=jax.ShapeDtypeStruct((N, 1), jnp.float32),
    )(p, h, wr_s, wt_s)


def kernel(x, edge_index, batch, W_rel, W_root):
    del batch
    src = edge_index[0]
    dst = edge_index[1]
    step = 4
    wr_s = W_rel[step - 1::step]
    wt_s = W_root[step - 1::step]

    p1 = _sc_hop(x, src, dst)
    h1 = _tc_mid(p1, x, W_rel, W_root)
    p2 = _sc_hop(h1, src, dst)
    out = _tc_final(p2, h1, wr_s, wt_s)
    return out[:, 0]


# speedup vs baseline: 10.8038x; 1.2376x over previous
import functools

import jax
import jax.numpy as jnp
from jax import lax
from jax.experimental import pallas as pl
from jax.experimental.pallas import tpu as pltpu
from jax.experimental.pallas import tpu_sc as plsc

N = 10000
E = 320000
D = 128
NC = 2
NS = 16
NW = NC * NS
EPT = E // NW
CHUNK = 128
NFULL = EPT // CHUNK
REM = EPT - NFULL * CHUNK
RPT = 624
TAIL = N - NS * RPT
ZR = 104


NB = 3
UNROLL = 6


def _sc_hop(h, src, dst, zeros):
    mesh = plsc.VectorSubcoreMesh(core_axis_name="c", subcore_axis_name="s",
                                  num_cores=NC, num_subcores=NS)

    @functools.partial(
        pl.kernel,
        mesh=mesh,
        out_type=jax.ShapeDtypeStruct((NC, N, D), jnp.float32),
        scratch_types=(
            [pltpu.VMEM((CHUNK,), jnp.int32) for _ in range(UNROLL)]
            + [pltpu.VMEM((CHUNK,), jnp.int32) for _ in range(UNROLL)]
            + [pltpu.VMEM((CHUNK, D), jnp.float32) for _ in range(NB)]
            + [pltpu.VMEM((REM,), jnp.int32)]
            + [pltpu.VMEM((REM,), jnp.int32)]
            + [pltpu.VMEM_SHARED((N, D), jnp.float32)]
            + [pltpu.SemaphoreType.DMA for _ in range(2 * UNROLL + 2 * NB + 1)]
        ),
    )
    def hop(h_hbm, src_hbm, dst_hbm, z_hbm, out_hbm, *refs):
        sbs = refs[0:UNROLL]
        dbs = refs[UNROLL:2 * UNROLL]
        rbs = refs[2 * UNROLL:2 * UNROLL + NB]
        src_r = refs[2 * UNROLL + NB]
        dst_r = refs[2 * UNROLL + NB + 1]
        agg = refs[2 * UNROLL + NB + 2]
        sems = refs[2 * UNROLL + NB + 3:]
        isems = sems[0:2 * UNROLL]
        gsems = sems[2 * UNROLL:2 * UNROLL + NB]
        ssems = sems[2 * UNROLL + NB:2 * UNROLL + 2 * NB]
        rsem = sems[2 * UNROLL + 2 * NB]

        c = lax.axis_index("c")
        s = lax.axis_index("s")
        wid = s * NC + c

        row0 = s * RPT
        pltpu.sync_copy(z_hbm.at[pl.ds(row0, RPT)], agg.at[pl.ds(row0, RPT)])

        @pl.when(s == NS - 1)
        def _zero_tail():
            pltpu.sync_copy(z_hbm.at[pl.ds(NS * RPT, TAIL)],
                            agg.at[pl.ds(NS * RPT, TAIL)])
        plsc.subcore_barrier()

        def idx_issue(j, k):
            base = wid * EPT + j * CHUNK
            a = pltpu.async_copy(src_hbm.at[pl.ds(base, CHUNK)], sbs[k],
                                 isems[2 * k])
            b = pltpu.async_copy(dst_hbm.at[pl.ds(base, CHUNK)], dbs[k],
                                 isems[2 * k + 1])
            return (a, b)

        def idx_wait(ab):
            ab[0].wait()
            ab[1].wait()

        def gather(k, p):
            return pltpu.async_copy(h_hbm.at[sbs[k]], rbs[p], gsems[p])

        def scat(k, p):
            return pltpu.async_copy(rbs[p], agg.at[dbs[k]], ssems[p], add=True)

        def body(i, carry):
            j = UNROLL * i
            i0 = idx_issue(j, 0)
            i1 = idx_issue(j + 1, 1)
            i2 = idx_issue(j + 2, 2)
            i3 = idx_issue(j + 3, 3)
            idx_wait(i0)
            g0 = gather(0, 0)
            idx_wait(i1)
            g1 = gather(1, 1)
            g0.wait()
            s0 = scat(0, 0)
            idx_wait(i2)
            g2 = gather(2, 2)
            g1.wait()
            s1 = scat(1, 1)
            s0.wait()
            i4 = idx_issue(j + 4, 4)
            idx_wait(i3)
            g3 = gather(3, 0)
            g2.wait()
            s2 = scat(2, 2)
            s1.wait()
            i5 = idx_issue(j + 5, 5)
            idx_wait(i4)
            g4 = gather(4, 1)
            g3.wait()
            s3 = scat(3, 0)
            s2.wait()
            idx_wait(i5)
            g5 = gather(5, 2)
            g4.wait()
            s4 = scat(4, 1)
            g5.wait()
            s5 = scat(5, 2)
            s3.wait()
            s4.wait()
            s5.wait()
            return carry
        lax.fori_loop(0, NFULL // UNROLL, body, 0)

        rbase = wid * EPT + NFULL * CHUNK
        pltpu.sync_copy(src_hbm.at[pl.ds(rbase, REM)], src_r)
        gr = pltpu.async_copy(h_hbm.at[src_r], rbs[0].at[pl.ds(0, REM)], rsem)
        pltpu.sync_copy(dst_hbm.at[pl.ds(rbase, REM)], dst_r)
        gr.wait()
        pltpu.sync_copy(rbs[0].at[pl.ds(0, REM)], agg.at[dst_r], add=True)

        plsc.subcore_barrier()
        pltpu.sync_copy(agg.at[pl.ds(row0, RPT)],
                        out_hbm.at[c, pl.ds(row0, RPT)])

        @pl.when(s == NS - 1)
        def _copy_tail():
            pltpu.sync_copy(agg.at[pl.ds(NS * RPT, TAIL)],
                            out_hbm.at[c, pl.ds(NS * RPT, TAIL)])

    return hop(h, src, dst, zeros)


_DOT = dict(preferred_element_type=jnp.float32,
            precision=lax.Precision.HIGHEST)
_BR = 1000


def _tc_mid(p, h, wr, wt):
    def body(p_ref, h_ref, wr_ref, wt_ref, o_ref):
        agg = p_ref[0] + p_ref[1]
        y = lax.dot_general(agg, wr_ref[...], (((1,), (1,)), ((), ())), **_DOT)
        y = y + lax.dot_general(h_ref[...], wt_ref[...],
                                (((1,), (1,)), ((), ())), **_DOT)
        o_ref[...] = jnp.maximum(y, 0.0)

    return pl.pallas_call(
        body,
        grid=(N // _BR,),
        in_specs=[
            pl.BlockSpec((NC, _BR, D), lambda i: (0, i, 0)),
            pl.BlockSpec((_BR, D), lambda i: (i, 0)),
            pl.BlockSpec((D, D), lambda i: (0, 0)),
            pl.BlockSpec((D, D), lambda i: (0, 0)),
        ],
        out_specs=pl.BlockSpec((_BR, D), lambda i: (i, 0)),
        out_shape=jax.ShapeDtypeStruct((N, D), jnp.float32),
    )(p, h, wr, wt)


def _tc_final(p, h, wr_s, wt_s):
    ksel = wr_s.shape[0]

    def body(p_ref, h_ref, wr_ref, wt_ref, o_ref):
        agg = p_ref[0] + p_ref[1]
        y = lax.dot_general(agg, wr_ref[...], (((1,), (1,)), ((), ())), **_DOT)
        y = y + lax.dot_general(h_ref[...], wt_ref[...],
                                (((1,), (1,)), ((), ())), **_DOT)
        o_ref[...] = jnp.sum(jnp.maximum(y, 0.0), axis=1, keepdims=True)

    return pl.pallas_call(
        body,
        grid=(N // _BR,),
        in_specs=[
            pl.BlockSpec((NC, _BR, D), lambda i: (0, i, 0)),
            pl.BlockSpec((_BR, D), lambda i: (i, 0)),
            pl.BlockSpec((ksel, D), lambda i: (0, 0)),
            pl.BlockSpec((ksel, D), lambda i: (0, 0)),
        ],
        out_specs=pl.BlockSpec((_BR, 1), lambda i: (i, 0)),
        out_shape=jax.ShapeDtypeStruct((N, 1), jnp.float32),
    )(p, h, wr_s, wt_s)


def kernel(x, edge_index, batch, W_rel, W_root):
    del batch
    src = edge_index[0]
    dst = edge_index[1]
    step = 4
    wr_s = W_rel[step - 1::step]
    wt_s = W_root[step - 1::step]

    zeros = jnp.zeros((N, D), jnp.float32)
    p1 = _sc_hop(x, src, dst, zeros)
    h1 = _tc_mid(p1, x, W_rel, W_root)
    p2 = _sc_hop(h1, src, dst, zeros)
    out = _tc_final(p2, h1, wr_s, wt_s)
    return out[:, 0]
